# baseline (speedup 1.0000x reference)
import functools

import jax
import jax.numpy as jnp
from jax import lax
from jax.experimental import pallas as pl
from jax.experimental.pallas import tpu as pltpu
from jax.experimental.pallas import tpu_sc as plsc

N = 10000
E = 320000
D = 128

NC = 2
NS = 16
NW = NC * NS

EPW = E // NW
EC = 80
CH = EPW // EC
SEC = 5
SCH = CH // SEC

NP = 10240
RPT = NP // NS
RC = EC
NRC = RPT // RC

def _zero_vmem(ref, nrows, ncols):
    def zrow(i, carry):
        for t in range(ncols // 16):
            ref[i, pl.ds(t * 16, 16)] = jnp.zeros((16,), jnp.float32)
        return carry
    lax.fori_loop(0, nrows, zrow, None)


def _deg_body(dst_hbm, out_hbm, dst_v, ones_v, stage_v, deg_s, sem):
    c = lax.axis_index("c")
    s = lax.axis_index("s")
    wid = s * NC + c

    pltpu.sync_copy(dst_hbm.at[wid], dst_v)
    for t in range(EC // 16):
        ones_v[pl.ds(t * 16, 16)] = jnp.full((16,), 1.0, jnp.float32)

    def zrow(i, carry):
        stage_v[pl.ds(i * 16, 16)] = jnp.zeros((16,), jnp.float32)
        return carry
    lax.fori_loop(0, RPT // 16, zrow, None)
    pltpu.sync_copy(stage_v, deg_s.at[pl.ds(s * RPT, RPT)])
    plsc.subcore_barrier()

    def body(j, carry):
        pltpu.async_copy(ones_v, deg_s.at[dst_v.at[j]], sem, add=True)
        return carry
    lax.fori_loop(0, CH, body, None)

    def drain(j, carry):
        pltpu.make_async_copy(ones_v, deg_s.at[dst_v.at[0]], sem).wait()
        return carry
    lax.fori_loop(0, CH, drain, None)
    plsc.subcore_barrier()

    pltpu.sync_copy(deg_s.at[pl.ds(s * RPT, RPT)],
                    out_hbm.at[c, pl.ds(s * RPT, RPT)])


@functools.cache
def _deg_kernel():
    return functools.partial(
        pl.kernel,
        out_type=jax.ShapeDtypeStruct((NC, NP), jnp.float32),
        mesh=plsc.VectorSubcoreMesh(
            core_axis_name="c", subcore_axis_name="s",
            num_cores=NC, num_subcores=NS),
        scratch_types=[
            pltpu.VMEM((CH, EC), jnp.int32),
            pltpu.VMEM((EC,), jnp.float32),
            pltpu.VMEM((RPT,), jnp.float32),
            pltpu.VMEM_SHARED((NP,), jnp.float32),
            pltpu.SemaphoreType.DMA,
        ],
    )(_deg_body)


def _edge_body(h_hbm, src_hbm, dst_hbm, out_hbm,
               src_v, dst_v, rows_a, rows_b, rows_c, acc_s,
               sem_a, sem_b, sem_c):
    c = lax.axis_index("c")
    s = lax.axis_index("s")
    wid = s * NC + c

    row0 = s * RPT

    @pl.when(c == 0)
    def _():
        @pl.when(s < NS - 1)
        def _():
            pltpu.sync_copy(h_hbm.at[pl.ds(row0, RPT)],
                            acc_s.at[pl.ds(row0, RPT)])

        @pl.when(s == NS - 1)
        def _():
            pltpu.sync_copy(h_hbm.at[pl.ds(row0, N - (NS - 1) * RPT)],
                            acc_s.at[pl.ds(row0, N - (NS - 1) * RPT)])

    @pl.when(c == 1)
    def _():
        _zero_vmem(rows_a, RC, D)

        def zcopy(k, carry):
            pltpu.sync_copy(rows_a, acc_s.at[pl.ds(row0 + k * RC, RC)])
            return carry
        lax.fori_loop(0, NRC, zcopy, None)
    plsc.subcore_barrier()

    def gather(j, buf, sem):
        pltpu.async_copy(h_hbm.at[src_v.at[j]], buf, sem)

    def gwait(buf, sem):
        pltpu.make_async_copy(h_hbm.at[src_v.at[0]], buf, sem).wait()

    def scatter(j, buf):
        pltpu.sync_copy(buf, acc_s.at[dst_v.at[j]], add=True)

    def section(sec, carry):
        pltpu.sync_copy(src_hbm.at[wid, sec], src_v)
        pltpu.sync_copy(dst_hbm.at[wid, sec], dst_v)
        gather(0, rows_a, sem_a)
        gather(1, rows_b, sem_b)

        def triple(t, carry):
            gwait(rows_a, sem_a)
            gather(3 * t + 2, rows_c, sem_c)
            scatter(3 * t, rows_a)
            gwait(rows_b, sem_b)
            gather(3 * t + 3, rows_a, sem_a)
            scatter(3 * t + 1, rows_b)
            gwait(rows_c, sem_c)
            gather(3 * t + 4, rows_b, sem_b)
            scatter(3 * t + 2, rows_c)
            return carry
        lax.fori_loop(0, (SCH - 4) // 3, triple, None)
        gwait(rows_a, sem_a)
        gather(SCH - 2, rows_c, sem_c)
        scatter(SCH - 4, rows_a)
        gwait(rows_b, sem_b)
        gather(SCH - 1, rows_a, sem_a)
        scatter(SCH - 3, rows_b)
        gwait(rows_c, sem_c)
        scatter(SCH - 2, rows_c)
        gwait(rows_a, sem_a)
        scatter(SCH - 1, rows_a)
        return carry

    lax.fori_loop(0, SEC, section, None)
    plsc.subcore_barrier()

    pltpu.sync_copy(acc_s.at[pl.ds(row0, RPT)], out_hbm.at[c, pl.ds(row0, RPT)])


@functools.cache
def _edge_kernel():
    return functools.partial(
        pl.kernel,
        out_type=jax.ShapeDtypeStruct((NC, NP, D), jnp.float32),
        mesh=plsc.VectorSubcoreMesh(
            core_axis_name="c", subcore_axis_name="s",
            num_cores=NC, num_subcores=NS),
        scratch_types=[
            pltpu.VMEM((SCH, EC), jnp.int32),
            pltpu.VMEM((SCH, EC), jnp.int32),
            pltpu.VMEM((EC, D), jnp.float32),
            pltpu.VMEM((EC, D), jnp.float32),
            pltpu.VMEM((EC, D), jnp.float32),
            pltpu.VMEM_SHARED((NP, D), jnp.float32),
            pltpu.SemaphoreType.DMA,
            pltpu.SemaphoreType.DMA,
            pltpu.SemaphoreType.DMA,
        ],
    )(_edge_body)


BLK = 2000


def _tc_a_body(x_ref, w_ref, degp_ref, out_ref):
    deg = degp_ref[:, 0:1] + degp_ref[:, 1:2] + 1.0
    dinv = lax.rsqrt(deg)
    h = jnp.dot(x_ref[...], w_ref[...], preferred_element_type=jnp.float32)
    out_ref[...] = h * dinv


def _tc_b_body(acc_ref, degp_ref, b_ref, w_ref, out_ref):
    deg = degp_ref[:, 0:1] + degp_ref[:, 1:2] + 1.0
    dinv = lax.rsqrt(deg)
    z = dinv * (acc_ref[0] + acc_ref[1]) + b_ref[...]
    z = jnp.maximum(z, 0.0)
    h2 = jnp.dot(z, w_ref[...], preferred_element_type=jnp.float32)
    out_ref[...] = h2 * dinv


def _tc_c_body(acc_ref, degp_ref, b_ref, out_ref):
    deg = degp_ref[:, 0:1] + degp_ref[:, 1:2] + 1.0
    dinv = lax.rsqrt(deg)
    out_ref[...] = dinv * (acc_ref[0] + acc_ref[1]) + b_ref[...]


_row_spec = pl.BlockSpec((BLK, D), lambda i: (i, 0))
_acc_spec = pl.BlockSpec((NC, BLK, D), lambda i: (0, i, 0))
_deg_spec = pl.BlockSpec((BLK, 2), lambda i: (i, 0))
_w_spec = pl.BlockSpec((D, D), lambda i: (0, 0))
_b_spec = pl.BlockSpec((1, D), lambda i: (0, 0))

_tc_a = pl.pallas_call(
    _tc_a_body,
    grid=(N // BLK,),
    in_specs=[_row_spec, _w_spec, _deg_spec],
    out_specs=_row_spec,
    out_shape=jax.ShapeDtypeStruct((N, D), jnp.float32),
)

_tc_b = pl.pallas_call(
    _tc_b_body,
    grid=(N // BLK,),
    in_specs=[_acc_spec, _deg_spec, _b_spec, _w_spec],
    out_specs=_row_spec,
    out_shape=jax.ShapeDtypeStruct((N, D), jnp.float32),
)

_tc_c = pl.pallas_call(
    _tc_c_body,
    grid=(N // BLK,),
    in_specs=[_acc_spec, _deg_spec, _b_spec],
    out_specs=_row_spec,
    out_shape=jax.ShapeDtypeStruct((N, D), jnp.float32),
)


def kernel(x, edge_index, W1, b1, W2, b2):
    src = edge_index[0].reshape(NW, SEC, SCH, EC)
    dst = edge_index[1].reshape(NW, SEC, SCH, EC)

    deg_parts = _deg_kernel()(dst.reshape(NW, CH, EC))
    degp = deg_parts[:, :N].T

    h1p = _tc_a(x, W1, degp)
    acc1 = _edge_kernel()(h1p, src, dst)
    h2p = _tc_b(acc1, degp, b1.reshape(1, D), W2)
    acc2 = _edge_kernel()(h2p, src, dst)
    out = _tc_c(acc2, degp, b2.reshape(1, D))
    return out


# speedup vs baseline: 1.0140x; 1.0140x over previous
import functools

import jax
import jax.numpy as jnp
from jax import lax
from jax.experimental import pallas as pl
from jax.experimental.pallas import tpu as pltpu
from jax.experimental.pallas import tpu_sc as plsc

N = 10000
E = 320000
D = 128

NC = 2
NS = 16
NW = NC * NS

EPW = E // NW
EC = 80
CH = EPW // EC
SEC = 5
SCH = CH // SEC

NP = 10240
RPT = NP // NS
RC = EC
NRC = RPT // RC

def _zero_vmem(ref, nrows, ncols):
    def zrow(i, carry):
        for t in range(ncols // 16):
            ref[i, pl.ds(t * 16, 16)] = jnp.zeros((16,), jnp.float32)
        return carry
    lax.fori_loop(0, nrows, zrow, None)


def _deg_body(dst_hbm, out_hbm, dst_v, ones_v, stage_v, deg_s, sem):
    c = lax.axis_index("c")
    s = lax.axis_index("s")
    wid = s * NC + c

    pltpu.sync_copy(dst_hbm.at[wid], dst_v)
    for t in range(EC // 16):
        ones_v[pl.ds(t * 16, 16)] = jnp.full((16,), 1.0, jnp.float32)

    def zrow(i, carry):
        stage_v[pl.ds(i * 16, 16)] = jnp.zeros((16,), jnp.float32)
        return carry
    lax.fori_loop(0, RPT // 16, zrow, None)
    pltpu.sync_copy(stage_v, deg_s.at[pl.ds(s * RPT, RPT)])
    plsc.subcore_barrier()

    def body(j, carry):
        pltpu.async_copy(ones_v, deg_s.at[dst_v.at[j]], sem, add=True)
        return carry
    lax.fori_loop(0, CH, body, None)

    def drain(j, carry):
        pltpu.make_async_copy(ones_v, deg_s.at[dst_v.at[0]], sem).wait()
        return carry
    lax.fori_loop(0, CH, drain, None)
    plsc.subcore_barrier()

    pltpu.sync_copy(deg_s.at[pl.ds(s * RPT, RPT)],
                    out_hbm.at[c, pl.ds(s * RPT, RPT)])


@functools.cache
def _deg_kernel():
    return functools.partial(
        pl.kernel,
        out_type=jax.ShapeDtypeStruct((NC, NP), jnp.float32),
        mesh=plsc.VectorSubcoreMesh(
            core_axis_name="c", subcore_axis_name="s",
            num_cores=NC, num_subcores=NS),
        scratch_types=[
            pltpu.VMEM((CH, EC), jnp.int32),
            pltpu.VMEM((EC,), jnp.float32),
            pltpu.VMEM((RPT,), jnp.float32),
            pltpu.VMEM_SHARED((NP,), jnp.float32),
            pltpu.SemaphoreType.DMA,
        ],
    )(_deg_body)


def _edge_body(h_hbm, src_hbm, dst_hbm, out_hbm,
               src_v, dst_v, rows_a, rows_b, rows_c, acc_s,
               sem_a, sem_b, sem_c):
    c = lax.axis_index("c")
    s = lax.axis_index("s")
    wid = s * NC + c

    row0 = s * RPT

    _zero_vmem(rows_a, RC, D)

    def zcopy(k, carry):
        pltpu.sync_copy(rows_a, acc_s.at[pl.ds(row0 + k * RC, RC)])
        return carry
    lax.fori_loop(0, NRC, zcopy, None)
    plsc.subcore_barrier()

    def gather(j, buf, sem):
        pltpu.async_copy(h_hbm.at[src_v.at[j]], buf, sem)

    def gwait(buf, sem):
        pltpu.make_async_copy(h_hbm.at[src_v.at[0]], buf, sem).wait()

    def scatter(j, buf):
        pltpu.sync_copy(buf, acc_s.at[dst_v.at[j]], add=True)

    def section(sec, carry):
        pltpu.sync_copy(src_hbm.at[wid, sec], src_v)
        pltpu.sync_copy(dst_hbm.at[wid, sec], dst_v)
        gather(0, rows_a, sem_a)
        gather(1, rows_b, sem_b)

        def triple(t, carry):
            gwait(rows_a, sem_a)
            gather(3 * t + 2, rows_c, sem_c)
            scatter(3 * t, rows_a)
            gwait(rows_b, sem_b)
            gather(3 * t + 3, rows_a, sem_a)
            scatter(3 * t + 1, rows_b)
            gwait(rows_c, sem_c)
            gather(3 * t + 4, rows_b, sem_b)
            scatter(3 * t + 2, rows_c)
            return carry
        lax.fori_loop(0, (SCH - 4) // 3, triple, None)
        gwait(rows_a, sem_a)
        gather(SCH - 2, rows_c, sem_c)
        scatter(SCH - 4, rows_a)
        gwait(rows_b, sem_b)
        gather(SCH - 1, rows_a, sem_a)
        scatter(SCH - 3, rows_b)
        gwait(rows_c, sem_c)
        scatter(SCH - 2, rows_c)
        gwait(rows_a, sem_a)
        scatter(SCH - 1, rows_a)
        return carry

    lax.fori_loop(0, SEC, section, None)
    plsc.subcore_barrier()

    pltpu.sync_copy(acc_s.at[pl.ds(row0, RPT)], out_hbm.at[c, pl.ds(row0, RPT)])


@functools.cache
def _edge_kernel():
    return functools.partial(
        pl.kernel,
        out_type=jax.ShapeDtypeStruct((NC, NP, D), jnp.float32),
        mesh=plsc.VectorSubcoreMesh(
            core_axis_name="c", subcore_axis_name="s",
            num_cores=NC, num_subcores=NS),
        scratch_types=[
            pltpu.VMEM((SCH, EC), jnp.int32),
            pltpu.VMEM((SCH, EC), jnp.int32),
            pltpu.VMEM((EC, D), jnp.float32),
            pltpu.VMEM((EC, D), jnp.float32),
            pltpu.VMEM((EC, D), jnp.float32),
            pltpu.VMEM_SHARED((NP, D), jnp.float32),
            pltpu.SemaphoreType.DMA,
            pltpu.SemaphoreType.DMA,
            pltpu.SemaphoreType.DMA,
        ],
    )(_edge_body)


BLK = 2000


def _tc_a_body(x_ref, w_ref, degp_ref, out_ref):
    deg = degp_ref[:, 0:1] + degp_ref[:, 1:2] + 1.0
    dinv = lax.rsqrt(deg)
    h = jnp.dot(x_ref[...], w_ref[...], preferred_element_type=jnp.float32)
    out_ref[...] = h * dinv


def _tc_b_body(acc_ref, hp_ref, degp_ref, b_ref, w_ref, out_ref):
    deg = degp_ref[:, 0:1] + degp_ref[:, 1:2] + 1.0
    dinv = lax.rsqrt(deg)
    z = dinv * (acc_ref[0] + acc_ref[1] + hp_ref[...]) + b_ref[...]
    z = jnp.maximum(z, 0.0)
    h2 = jnp.dot(z, w_ref[...], preferred_element_type=jnp.float32)
    out_ref[...] = h2 * dinv


def _tc_c_body(acc_ref, hp_ref, degp_ref, b_ref, out_ref):
    deg = degp_ref[:, 0:1] + degp_ref[:, 1:2] + 1.0
    dinv = lax.rsqrt(deg)
    out_ref[...] = dinv * (acc_ref[0] + acc_ref[1] + hp_ref[...]) + b_ref[...]


_row_spec = pl.BlockSpec((BLK, D), lambda i: (i, 0))
_acc_spec = pl.BlockSpec((NC, BLK, D), lambda i: (0, i, 0))
_deg_spec = pl.BlockSpec((BLK, 2), lambda i: (i, 0))
_w_spec = pl.BlockSpec((D, D), lambda i: (0, 0))
_b_spec = pl.BlockSpec((1, D), lambda i: (0, 0))

_tc_a = pl.pallas_call(
    _tc_a_body,
    grid=(N // BLK,),
    in_specs=[_row_spec, _w_spec, _deg_spec],
    out_specs=_row_spec,
    out_shape=jax.ShapeDtypeStruct((N, D), jnp.float32),
)

_tc_b = pl.pallas_call(
    _tc_b_body,
    grid=(N // BLK,),
    in_specs=[_acc_spec, _row_spec, _deg_spec, _b_spec, _w_spec],
    out_specs=_row_spec,
    out_shape=jax.ShapeDtypeStruct((N, D), jnp.float32),
)

_tc_c = pl.pallas_call(
    _tc_c_body,
    grid=(N // BLK,),
    in_specs=[_acc_spec, _row_spec, _deg_spec, _b_spec],
    out_specs=_row_spec,
    out_shape=jax.ShapeDtypeStruct((N, D), jnp.float32),
)


def kernel(x, edge_index, W1, b1, W2, b2):
    src = edge_index[0].reshape(NW, SEC, SCH, EC)
    dst = edge_index[1].reshape(NW, SEC, SCH, EC)

    deg_parts = _deg_kernel()(dst.reshape(NW, CH, EC))
    degp = deg_parts[:, :N].T

    h1p = _tc_a(x, W1, degp)
    acc1 = _edge_kernel()(h1p, src, dst)
    h2p = _tc_b(acc1, h1p, degp, b1.reshape(1, D), W2)
    acc2 = _edge_kernel()(h2p, src, dst)
    out = _tc_c(acc2, h2p, degp, b2.reshape(1, D))
    return out


# speedup vs baseline: 1.0243x; 1.0102x over previous
import functools

import jax
import jax.numpy as jnp
from jax import lax
from jax.experimental import pallas as pl
from jax.experimental.pallas import tpu as pltpu
from jax.experimental.pallas import tpu_sc as plsc

N = 10000
E = 320000
D = 128

NC = 2
NS = 16
NW = NC * NS

EPW = E // NW
EC = 80
CH = EPW // EC
SEC = 5
SCH = CH // SEC

NP = 10240
RPT = NP // NS
RC = EC
NRC = RPT // RC

def _zero_vmem(ref, nrows, ncols):
    def zrow(i, carry):
        for t in range(ncols // 16):
            ref[i, pl.ds(t * 16, 16)] = jnp.zeros((16,), jnp.float32)
        return carry
    lax.fori_loop(0, nrows, zrow, None)


def _deg_body(dst_hbm, out_hbm, dst_v, ones_v, stage_v, deg_s, sem):
    c = lax.axis_index("c")
    s = lax.axis_index("s")
    wid = s * NC + c

    pltpu.sync_copy(dst_hbm.at[wid], dst_v)
    for t in range(EC // 16):
        ones_v[pl.ds(t * 16, 16)] = jnp.full((16,), 1.0, jnp.float32)

    def zrow(i, carry):
        stage_v[pl.ds(i * 16, 16)] = jnp.zeros((16,), jnp.float32)
        return carry
    lax.fori_loop(0, RPT // 16, zrow, None)
    pltpu.sync_copy(stage_v, deg_s.at[pl.ds(s * RPT, RPT)])
    plsc.subcore_barrier()

    def body(j, carry):
        pltpu.async_copy(ones_v, deg_s.at[dst_v.at[j]], sem, add=True)
        return carry
    lax.fori_loop(0, CH, body, None)

    def drain(j, carry):
        pltpu.make_async_copy(ones_v, deg_s.at[dst_v.at[0]], sem).wait()
        return carry
    lax.fori_loop(0, CH, drain, None)
    plsc.subcore_barrier()

    pltpu.sync_copy(deg_s.at[pl.ds(s * RPT, RPT)],
                    out_hbm.at[c, pl.ds(s * RPT, RPT)])


@functools.cache
def _deg_kernel():
    return functools.partial(
        pl.kernel,
        out_type=jax.ShapeDtypeStruct((NC, NP), jnp.float32),
        mesh=plsc.VectorSubcoreMesh(
            core_axis_name="c", subcore_axis_name="s",
            num_cores=NC, num_subcores=NS),
        scratch_types=[
            pltpu.VMEM((CH, EC), jnp.int32),
            pltpu.VMEM((EC,), jnp.float32),
            pltpu.VMEM((RPT,), jnp.float32),
            pltpu.VMEM_SHARED((NP,), jnp.float32),
            pltpu.SemaphoreType.DMA,
        ],
    )(_deg_body)


def _edge_body(h_hbm, src_hbm, dst_hbm, out_hbm,
               src_v, dst_v, rows_a, rows_b, rows_c, acc_s,
               sem_a, sem_b, sem_c):
    c = lax.axis_index("c")
    s = lax.axis_index("s")
    wid = s * NC + c

    row0 = s * RPT

    _zero_vmem(rows_a, RC, D)

    def zcopy(k, carry):
        pltpu.sync_copy(rows_a, acc_s.at[pl.ds(row0 + k * RC, RC)])
        return carry
    lax.fori_loop(0, NRC, zcopy, None)
    plsc.subcore_barrier()

    def gather(j, buf, sem):
        pltpu.async_copy(h_hbm.at[src_v.at[j]], buf, sem)

    def gwait(buf, sem):
        pltpu.make_async_copy(h_hbm.at[src_v.at[0]], buf, sem).wait()

    def scatter(j, buf):
        pltpu.sync_copy(buf, acc_s.at[dst_v.at[j]], add=True)

    def section(sec, carry):
        pltpu.sync_copy(src_hbm.at[wid, sec], src_v)
        pltpu.sync_copy(dst_hbm.at[wid, sec], dst_v)
        gather(0, rows_a, sem_a)
        gather(1, rows_b, sem_b)

        def triple(t, carry):
            gwait(rows_a, sem_a)
            gather(3 * t + 2, rows_c, sem_c)
            scatter(3 * t, rows_a)
            gwait(rows_b, sem_b)
            gather(3 * t + 3, rows_a, sem_a)
            scatter(3 * t + 1, rows_b)
            gwait(rows_c, sem_c)
            gather(3 * t + 4, rows_b, sem_b)
            scatter(3 * t + 2, rows_c)
            return carry
        lax.fori_loop(0, (SCH - 4) // 3, triple, None)
        gwait(rows_a, sem_a)
        gather(SCH - 2, rows_c, sem_c)
        scatter(SCH - 4, rows_a)
        gwait(rows_b, sem_b)
        gather(SCH - 1, rows_a, sem_a)
        scatter(SCH - 3, rows_b)
        gwait(rows_c, sem_c)
        scatter(SCH - 2, rows_c)
        gwait(rows_a, sem_a)
        scatter(SCH - 1, rows_a)
        return carry

    lax.fori_loop(0, SEC, section, None)
    plsc.subcore_barrier()

    pltpu.sync_copy(acc_s.at[pl.ds(row0, RPT)], out_hbm.at[c, pl.ds(row0, RPT)])


@functools.cache
def _edge_kernel():
    return functools.partial(
        pl.kernel,
        out_type=jax.ShapeDtypeStruct((NC, NP, D), jnp.float32),
        mesh=plsc.VectorSubcoreMesh(
            core_axis_name="c", subcore_axis_name="s",
            num_cores=NC, num_subcores=NS),
        scratch_types=[
            pltpu.VMEM((SCH, EC), jnp.int32),
            pltpu.VMEM((SCH, EC), jnp.int32),
            pltpu.VMEM((EC, D), jnp.float32),
            pltpu.VMEM((EC, D), jnp.float32),
            pltpu.VMEM((EC, D), jnp.float32),
            pltpu.VMEM_SHARED((NP, D), jnp.float32),
            pltpu.SemaphoreType.DMA,
            pltpu.SemaphoreType.DMA,
            pltpu.SemaphoreType.DMA,
        ],
    )(_edge_body)


BLK = 5000


def _tc_a_body(x_ref, w_ref, degp_ref, out_ref):
    deg = degp_ref[:, 0:1] + degp_ref[:, 1:2] + 1.0
    dinv = lax.rsqrt(deg)
    h = jnp.dot(x_ref[...], w_ref[...], preferred_element_type=jnp.float32)
    out_ref[...] = h * dinv


def _tc_b_body(acc_ref, hp_ref, degp_ref, b_ref, w_ref, out_ref):
    deg = degp_ref[:, 0:1] + degp_ref[:, 1:2] + 1.0
    dinv = lax.rsqrt(deg)
    z = dinv * (acc_ref[0] + acc_ref[1] + hp_ref[...]) + b_ref[...]
    z = jnp.maximum(z, 0.0)
    h2 = jnp.dot(z, w_ref[...], preferred_element_type=jnp.float32)
    out_ref[...] = h2 * dinv


def _tc_c_body(acc_ref, hp_ref, degp_ref, b_ref, out_ref):
    deg = degp_ref[:, 0:1] + degp_ref[:, 1:2] + 1.0
    dinv = lax.rsqrt(deg)
    out_ref[...] = dinv * (acc_ref[0] + acc_ref[1] + hp_ref[...]) + b_ref[...]


_row_spec = pl.BlockSpec((BLK, D), lambda i: (i, 0))
_acc_spec = pl.BlockSpec((NC, BLK, D), lambda i: (0, i, 0))
_deg_spec = pl.BlockSpec((BLK, 2), lambda i: (i, 0))
_w_spec = pl.BlockSpec((D, D), lambda i: (0, 0))
_b_spec = pl.BlockSpec((1, D), lambda i: (0, 0))

_tc_a = pl.pallas_call(
    _tc_a_body,
    grid=(N // BLK,),
    in_specs=[_row_spec, _w_spec, _deg_spec],
    out_specs=_row_spec,
    out_shape=jax.ShapeDtypeStruct((N, D), jnp.float32),
)

_tc_b = pl.pallas_call(
    _tc_b_body,
    grid=(N // BLK,),
    in_specs=[_acc_spec, _row_spec, _deg_spec, _b_spec, _w_spec],
    out_specs=_row_spec,
    out_shape=jax.ShapeDtypeStruct((N, D), jnp.float32),
)

_tc_c = pl.pallas_call(
    _tc_c_body,
    grid=(N // BLK,),
    in_specs=[_acc_spec, _row_spec, _deg_spec, _b_spec],
    out_specs=_row_spec,
    out_shape=jax.ShapeDtypeStruct((N, D), jnp.float32),
)


def kernel(x, edge_index, W1, b1, W2, b2):
    src = edge_index[0].reshape(NW, SEC, SCH, EC)
    dst = edge_index[1].reshape(NW, SEC, SCH, EC)

    deg_parts = _deg_kernel()(dst.reshape(NW, CH, EC))
    degp = deg_parts[:, :N].T

    h1p = _tc_a(x, W1, degp)
    acc1 = _edge_kernel()(h1p, src, dst)
    h2p = _tc_b(acc1, h1p, degp, b1.reshape(1, D), W2)
    acc2 = _edge_kernel()(h2p, src, dst)
    out = _tc_c(acc2, h2p, degp, b2.reshape(1, D))
    return out
